# baseline (speedup 1.0000x reference)
import jax
import jax.numpy as jnp
from jax.experimental import pallas as pl
from jax.experimental.pallas import tpu as pltpu

_TEMPERATURE = 0.25
_THRESHOLD = 0.5
_S = 4
_NT = 4


def _dot_nt(a, b):
    return jax.lax.dot_general(a, b, (((1,), (1,)), ((), ())),
                               preferred_element_type=jnp.float32)


def _dot_nn(a, b):
    return jax.lax.dot_general(a, b, (((1,), (0,)), ((), ())),
                               preferred_element_type=jnp.float32)


def _body(q_ref, wq_ref, wr_ref, k0_ref, k1_ref, v0_ref, v1_ref, v2_ref,
          v3_ref, out_ref, w_ref, acc_ref):
    t = pl.program_id(1)
    n_tiles, _, mt = w_ref.shape
    kh = k0_ref.shape[0]

    @pl.when(t == 0)
    def _score():
        q = q_ref[0]
        qe = _dot_nt(q, wq_ref[...])
        e = jnp.exp(qe * (1.0 / _TEMPERATURE))
        denom = 1.0 + jnp.sum(e, axis=-1, keepdims=True)
        num = jnp.concatenate([e, jnp.ones_like(denom)], axis=-1)
        qs = jnp.sqrt(num / denom)

        def s8_of(k_ref):
            s = _dot_nt(qs, k_ref[...])
            s2 = s * s
            s4 = s2 * s2
            return s4 * s4

        s8_0 = s8_of(k0_ref)
        s8_1 = s8_of(k1_ref)
        mx = jnp.maximum(jnp.max(s8_0, axis=-1, keepdims=True),
                         jnp.max(s8_1, axis=-1, keepdims=True))
        thr = jnp.where(mx < _THRESHOLD, 0.9 * mx, _THRESHOLD)
        m0 = jnp.where(s8_0 < thr, 0.0, s8_0)
        m1 = jnp.where(s8_1 < thr, 0.0, s8_1)
        wsum = (jnp.sum(m0, axis=-1, keepdims=True)
                + jnp.sum(m1, axis=-1, keepdims=True) + 1e-9)
        w0 = m0 / wsum
        w1 = m1 / wsum
        for i in range(n_tiles):
            off = i * mt
            if off < kh:
                w_ref[i] = w0[:, off:off + mt]
            else:
                w_ref[i] = w1[:, off - kh:off - kh + mt]
        acc_ref[...] = jnp.zeros_like(acc_ref)

    v_refs = (v0_ref, v1_ref, v2_ref, v3_ref)
    contrib = _dot_nn(w_ref[t * _S], v_refs[0][0])
    for i in range(1, _S):
        contrib += _dot_nn(w_ref[t * _S + i], v_refs[i][0])
    acc_ref[...] += contrib

    @pl.when(t == _NT - 1)
    def _finish():
        out_ref[0] = _dot_nt(acc_ref[...], wr_ref[...])


def kernel(queries, W_query, W_read, memory_keys, memory_values):
    B, Q, QD = queries.shape
    VD, ED = W_read.shape
    M, EDp1 = memory_keys.shape
    mt = M // (_S * _NT)
    kh = M // 2
    vspecs = [
        pl.BlockSpec((1, mt, ED), lambda b, t, i=i: (b, t * _S + i, 0))
        for i in range(_S)
    ]
    return pl.pallas_call(
        _body,
        grid=(B, _NT),
        in_specs=[
            pl.BlockSpec((1, Q, QD), lambda b, t: (b, 0, 0)),
            pl.BlockSpec((ED, QD), lambda b, t: (0, 0)),
            pl.BlockSpec((VD, ED), lambda b, t: (0, 0)),
            pl.BlockSpec((kh, EDp1), lambda b, t: (0, 0)),
            pl.BlockSpec((kh, EDp1), lambda b, t: (1, 0)),
            *vspecs,
        ],
        out_specs=pl.BlockSpec((1, Q, VD), lambda b, t: (b, 0, 0)),
        out_shape=jax.ShapeDtypeStruct((B, Q, VD), jnp.float32),
        scratch_shapes=[
            pltpu.VMEM((_S * _NT, Q, mt), jnp.float32),
            pltpu.VMEM((Q, ED), jnp.float32),
        ],
        compiler_params=pltpu.CompilerParams(
            dimension_semantics=("arbitrary", "arbitrary")),
    )(queries, W_query, W_read, memory_keys, memory_keys,
      memory_values, memory_values, memory_values, memory_values)


# speedup vs baseline: 3.6204x; 3.6204x over previous
import jax
import jax.numpy as jnp
from jax.experimental import pallas as pl
from jax.experimental.pallas import tpu as pltpu

_TEMPERATURE = 0.25
_THRESHOLD = 0.5


def _dot_nt(a, b):
    return jax.lax.dot_general(a, b, (((1,), (1,)), ((), ())),
                               preferred_element_type=jnp.float32)


def _dot_nn(a, b):
    return jax.lax.dot_general(a, b, (((1,), (0,)), ((), ())),
                               preferred_element_type=jnp.float32)


def _body(q_ref, wq_ref, wr_ref, keys_ref, vals_ref, out_ref):
    q = q_ref[0]
    qe = _dot_nt(q, wq_ref[...])
    e = jnp.exp(qe * (1.0 / _TEMPERATURE))
    denom = 1.0 + jnp.sum(e, axis=-1, keepdims=True)
    num = jnp.concatenate([e, jnp.ones_like(denom)], axis=-1)
    qs = jnp.sqrt(num / denom)
    scores = _dot_nn(qs, keys_ref[...])
    s2 = scores * scores
    s4 = s2 * s2
    s8 = s4 * s4
    mx = jnp.max(s8, axis=-1, keepdims=True)
    thr = jnp.where(mx < _THRESHOLD, 0.9 * mx, _THRESHOLD)
    masked = jnp.where(s8 < thr, 0.0, s8)
    w = masked / (jnp.sum(masked, axis=-1, keepdims=True) + 1e-9)
    read = _dot_nt(w, vals_ref[0])
    out_ref[0] = _dot_nn(read, wr_ref[...])


def kernel(queries, W_query, W_read, memory_keys, memory_values):
    B, Q, QD = queries.shape
    VD, ED = W_read.shape
    M, EDp1 = memory_keys.shape
    keys_t = memory_keys.T
    vals_t = memory_values.transpose(0, 2, 1)
    wr_t = W_read.T
    return pl.pallas_call(
        _body,
        grid=(B,),
        in_specs=[
            pl.BlockSpec((1, Q, QD), lambda b: (b, 0, 0)),
            pl.BlockSpec((ED, QD), lambda b: (0, 0)),
            pl.BlockSpec((ED, VD), lambda b: (0, 0)),
            pl.BlockSpec((EDp1, M), lambda b: (0, 0)),
            pl.BlockSpec((1, ED, M), lambda b: (b, 0, 0)),
        ],
        out_specs=pl.BlockSpec((1, Q, VD), lambda b: (b, 0, 0)),
        out_shape=jax.ShapeDtypeStruct((B, Q, VD), jnp.float32),
        compiler_params=pltpu.CompilerParams(
            dimension_semantics=("arbitrary",)),
    )(queries, W_query, wr_t, keys_t, vals_t)


# speedup vs baseline: 3.6776x; 1.0158x over previous
import jax
import jax.numpy as jnp
from jax.experimental import pallas as pl
from jax.experimental.pallas import tpu as pltpu

_TEMPERATURE = 0.25
_THRESHOLD = 0.5


def _dot_nt(a, b):
    return jax.lax.dot_general(a, b, (((1,), (1,)), ((), ())),
                               preferred_element_type=jnp.float32)


def _dot_nn(a, b):
    return jax.lax.dot_general(a, b, (((1,), (0,)), ((), ())),
                               preferred_element_type=jnp.float32)


_SK = 2
_SV = 4


def _body(q_ref, wq_ref, wr_ref, *rest):
    keys_refs = rest[:_SK]
    vals_refs = rest[_SK:_SK + _SV]
    out_ref = rest[_SK + _SV]
    q = q_ref[0]
    qe = _dot_nt(q, wq_ref[...])
    e = jnp.exp(qe * (1.0 / _TEMPERATURE))
    denom = 1.0 + jnp.sum(e, axis=-1, keepdims=True)
    num = jnp.concatenate([e, jnp.ones_like(denom)], axis=-1)
    qs = jnp.sqrt(num / denom)
    scores = [_dot_nn(qs, k[...]) for k in keys_refs]

    def pow8(s):
        s2 = s * s
        s4 = s2 * s2
        return s4 * s4

    s8 = [pow8(s) for s in scores]
    mx = s8[0].max(axis=-1, keepdims=True)
    for s in s8[1:]:
        mx = jnp.maximum(mx, s.max(axis=-1, keepdims=True))
    thr = jnp.where(mx < _THRESHOLD, 0.9 * mx, _THRESHOLD)
    masked = [jnp.where(s < thr, 0.0, s) for s in s8]
    wsum = masked[0].sum(axis=-1, keepdims=True)
    for ms in masked[1:]:
        wsum = wsum + ms.sum(axis=-1, keepdims=True)
    mh = masked[0].shape[1]
    vh = vals_refs[0].shape[2]
    read = None
    for i, v in enumerate(vals_refs):
        wslice = masked[(i * vh) // mh][:, (i * vh) % mh:(i * vh) % mh + vh]
        c = _dot_nt(wslice, v[0])
        read = c if read is None else read + c
    read = read / (wsum + 1e-9)
    out_ref[0] = _dot_nn(read, wr_ref[...])


def kernel(queries, W_query, W_read, memory_keys, memory_values):
    B, Q, QD = queries.shape
    VD, ED = W_read.shape
    M, EDp1 = memory_keys.shape
    keys_t = memory_keys.T
    vals_t = memory_values.transpose(0, 2, 1)
    wr_t = W_read.T
    kh = M // _SK
    vh = M // _SV
    kspecs = [
        pl.BlockSpec((EDp1, kh), lambda b, i=i: (0, i)) for i in range(_SK)
    ]
    vspecs = [
        pl.BlockSpec((1, ED, vh), lambda b, i=i: (b, 0, i)) for i in range(_SV)
    ]
    return pl.pallas_call(
        _body,
        grid=(B,),
        in_specs=[
            pl.BlockSpec((1, Q, QD), lambda b: (b, 0, 0)),
            pl.BlockSpec((ED, QD), lambda b: (0, 0)),
            pl.BlockSpec((ED, VD), lambda b: (0, 0)),
            *kspecs,
            *vspecs,
        ],
        out_specs=pl.BlockSpec((1, Q, VD), lambda b: (b, 0, 0)),
        out_shape=jax.ShapeDtypeStruct((B, Q, VD), jnp.float32),
        compiler_params=pltpu.CompilerParams(
            dimension_semantics=("arbitrary",)),
    )(queries, W_query, wr_t, *([keys_t] * _SK), *([vals_t] * _SV))
